# baseline (speedup 1.0000x reference)
import functools

import jax
import jax.numpy as jnp
from jax import lax
from jax.experimental import pallas as pl
from jax.experimental.pallas import tpu as pltpu
from jax.experimental.pallas import tpu_sc as plsc

N = 10000
E = 320000
D = 128
H = 128

NC = 2
NS = 16
NW = NC * NS
EPT = E // NW
CH = 40
NCH = EPT // CH
ROWS_PT = N // NS
BLK = 2000
NBUF = 5
NFULL = (NCH // NBUF - 1) * NBUF


def _mesh():
    return plsc.VectorSubcoreMesh(
        core_axis_name="c", subcore_axis_name="s", num_cores=NC, num_subcores=NS
    )


_SC_PARAMS = pltpu.CompilerParams(
    needs_layout_passes=False, use_tc_tiling_on_sc=False
)


def _hist_body(ei_hbm, aux_hbm, hist_hbm, srcv, dstv, hist_o, hist_i, ridx, shd):
    cid = lax.axis_index("c")
    sid = lax.axis_index("s")
    wid = cid * NS + sid
    base = wid * EPT
    pltpu.sync_copy(ei_hbm.at[0, pl.ds(base, EPT)], srcv)
    pltpu.sync_copy(ei_hbm.at[1, pl.ds(base, EPT)], dstv)
    pltpu.sync_copy(aux_hbm, ridx)

    zeros16 = jnp.zeros((16,), jnp.float32)

    def zbody(i, c):
        for u in range(5):
            o = (i * 5 + u) * 16
            hist_o[0, pl.ds(o, 16)] = zeros16
            hist_i[0, pl.ds(o, 16)] = zeros16
        return c

    lax.fori_loop(0, N // 80, zbody, 0)
    @pl.when(sid < NS - 6)
    def _():
        row = sid // 5
        pltpu.sync_copy(
            hist_o.at[0, pl.ds(0, 1000)],
            shd.at[row, pl.ds((sid % 5) * 2000, 1000)],
        )
        pltpu.sync_copy(
            hist_i.at[0, pl.ds(0, 1000)],
            shd.at[row, pl.ds((sid % 5) * 2000 + 1000, 1000)],
        )

    plsc.subcore_barrier()

    ones16 = jnp.ones((16,), jnp.float32)
    zidx16 = jnp.zeros((16,), jnp.int32)

    def ebody(i, c):
        for u in range(5):
            o = (i * 5 + u) * 16
            s = srcv[pl.ds(o, 16)]
            plsc.addupdate_scatter(hist_o, [zidx16, s], ones16)
            d = dstv[pl.ds(o, 16)]
            plsc.addupdate_scatter(hist_i, [zidx16, d], ones16)
        return c

    lax.fori_loop(0, EPT // 80, ebody, 0)

    pltpu.sync_copy(hist_o, shd.at[ridx.at[pl.ds(0, 1)]], add=True)
    pltpu.sync_copy(hist_i, shd.at[ridx.at[pl.ds(8, 1)]], add=True)
    plsc.subcore_barrier()

    @pl.when(sid == 0)
    def _():
        pltpu.sync_copy(shd.at[0], hist_hbm.at[0, cid])
        pltpu.sync_copy(shd.at[1], hist_hbm.at[1, cid])


def _hist_call(edge_index, aux):
    f = pl.kernel(
        _hist_body,
        out_type=jax.ShapeDtypeStruct((2, NC, N), jnp.float32),
        mesh=_mesh(),
        scratch_types=[
            pltpu.VMEM((EPT,), jnp.int32),
            pltpu.VMEM((EPT,), jnp.int32),
            pltpu.VMEM((1, N), jnp.float32),
            pltpu.VMEM((1, N), jnp.float32),
            pltpu.VMEM((16,), jnp.int32),
            pltpu.VMEM_SHARED((2, N), jnp.float32),
        ],
        compiler_params=_SC_PARAMS,
    )
    return f(edge_index, aux)


MARGIN = 96.0


def _scatter_body(y_hbm, ei_hbm, part_hbm, srcv, dstv, rows, *rest):
    sems = rest[:NBUF]
    dsem = rest[NBUF]
    zsem = rest[NBUF + 1]
    acc = rest[NBUF + 2]
    cid = lax.axis_index("c")
    sid = lax.axis_index("s")
    wid = cid * NS + sid
    base = wid * EPT
    r0 = sid * ROWS_PT
    pltpu.sync_copy(ei_hbm.at[0, pl.ds(base, EPT)], srcv)
    def stage(r, c):
        pltpu.async_copy(ei_hbm.at[1, pl.ds(base + r * CH, CH)], dstv.at[r], dsem)
        return c

    lax.fori_loop(0, NCH, stage, 0)
    zeros32 = jnp.zeros((32,), jnp.int16)

    def zb(i, c):
        for u in range(H // 32):
            rows[0, i, pl.ds(u * 32, 32)] = zeros32
        return c

    lax.fori_loop(0, CH, zb, 0)
    for t in range(ROWS_PT // CH):
        pltpu.async_copy(rows.at[0], acc.at[pl.ds(r0 + t * CH, CH)], zsem)
    pltpu.async_copy(
        rows.at[0, pl.ds(0, ROWS_PT % CH)],
        acc.at[pl.ds(r0 + (ROWS_PT // CH) * CH, ROWS_PT % CH)],
        zsem,
    )
    for t in range(ROWS_PT // CH):
        pltpu.make_async_copy(rows.at[0], acc.at[pl.ds(r0 + t * CH, CH)], zsem).wait()
    pltpu.make_async_copy(
        rows.at[0, pl.ds(0, ROWS_PT % CH)],
        acc.at[pl.ds(r0 + (ROWS_PT // CH) * CH, ROWS_PT % CH)],
        zsem,
    ).wait()

    def drain(r, c):
        pltpu.make_async_copy(
            ei_hbm.at[1, pl.ds(base + r * CH, CH)], dstv.at[r], dsem
        ).wait()
        return c

    lax.fori_loop(0, NCH, drain, 0)
    plsc.subcore_barrier()

    for b in range(NBUF):
        pltpu.async_copy(y_hbm.at[srcv.at[pl.ds(b * CH, CH)]], rows.at[b], sems[b])

    def body(g, c):
        for b in range(NBUF):
            j = g * NBUF + b
            pltpu.make_async_copy(
                y_hbm.at[srcv.at[pl.ds(j * CH, CH)]], rows.at[b], sems[b]
            ).wait()
            pltpu.sync_copy(rows.at[b], acc.at[dstv.at[j]], add=True)
            pltpu.async_copy(
                y_hbm.at[srcv.at[pl.ds((j + NBUF) * CH, CH)]], rows.at[b], sems[b]
            )
        return c

    lax.fori_loop(0, NCH // NBUF - 1, body, 0)
    for b in range(NBUF):
        j = NFULL + b
        pltpu.make_async_copy(
            y_hbm.at[srcv.at[pl.ds(j * CH, CH)]], rows.at[b], sems[b]
        ).wait()
        pltpu.sync_copy(rows.at[b], acc.at[dstv.at[j]], add=True)
    for j in range(NFULL + NBUF, NCH):
        b = j % NBUF
        pltpu.async_copy(y_hbm.at[srcv.at[pl.ds(j * CH, CH)]], rows.at[b], sems[b]).wait()
        pltpu.sync_copy(rows.at[b], acc.at[dstv.at[j]], add=True)

    plsc.subcore_barrier()
    pltpu.sync_copy(acc.at[pl.ds(r0, ROWS_PT)], part_hbm.at[cid, pl.ds(r0, ROWS_PT)])


def _scatter_call(y, edge_index):
    f = pl.kernel(
        _scatter_body,
        out_type=jax.ShapeDtypeStruct((NC, N, H), jnp.int16),
        mesh=_mesh(),
        scratch_types=[
            pltpu.VMEM((EPT,), jnp.int32),
            pltpu.VMEM((NCH, CH), jnp.int32),
            pltpu.VMEM((NBUF, CH, H), jnp.int16),
        ]
        + [pltpu.SemaphoreType.DMA] * (NBUF + 2)
        + [pltpu.VMEM_SHARED((N, H), jnp.int16)],
        compiler_params=_SC_PARAMS,
    )
    return f(y, edge_index)


def _nrm_body(hist_ref, nrm_ref):
    ones_w = jnp.ones((NC, 1), jnp.float32)
    dn = (((0,), (0,)), ((), ()))
    od = lax.dot_general(hist_ref[0], ones_w, dn, preferred_element_type=jnp.float32)
    idg = lax.dot_general(hist_ref[1], ones_w, dn, preferred_element_type=jnp.float32)
    onrm = lax.rsqrt(jnp.maximum(od, 1.0))
    inrm = lax.rsqrt(jnp.maximum(idg, 1.0))
    nrm_ref[...] = jnp.concatenate([onrm, inrm], axis=1)


def _nrm_call(hist):
    return pl.pallas_call(
        _nrm_body,
        grid=(1,),
        in_specs=[pl.BlockSpec((2, NC, N), lambda i: (0, 0, 0))],
        out_specs=pl.BlockSpec((N, 2), lambda i: (0, 0)),
        out_shape=jax.ShapeDtypeStruct((N, 2), jnp.float32),
    )(hist)


def _blockmax(y):
    return jnp.max(jnp.abs(y).reshape(BLK // 8, 8, H), axis=0)


def _quantize(ybuf_ref, max_ref, i, q_ref):
    m = jnp.max(max_ref[...])
    scale = 32767.0 / (MARGIN * jnp.maximum(m, 1e-30))
    y = ybuf_ref[pl.ds(i * BLK, BLK), :]
    q_ref[...] = jnp.rint(y * scale).astype(jnp.int16)


def _dequant_agg(part_ref, max_ref):
    m = jnp.max(max_ref[...])
    inv = (MARGIN * jnp.maximum(m, 1e-30)) / 32767.0
    return (
        part_ref[0].astype(jnp.float32) + part_ref[1].astype(jnp.float32)
    ) * inv


def _tc1_body(x_ref, w_ref, nrm_ref, q_ref, max_ref, ybuf_ref):
    p = pl.program_id(0)
    i = pl.program_id(1)

    @pl.when(p == 0)
    def _():
        z = jnp.dot(x_ref[...], w_ref[...], preferred_element_type=jnp.float32)
        y = nrm_ref[:, 0:1] * z
        ybuf_ref[pl.ds(i * BLK, BLK), :] = y
        mb = _blockmax(y)

        @pl.when(i == 0)
        def _():
            max_ref[...] = mb

        @pl.when(i > 0)
        def _():
            max_ref[...] = jnp.maximum(max_ref[...], mb)

    @pl.when(p == 1)
    def _():
        _quantize(ybuf_ref, max_ref, i, q_ref)


def _tc1_call(x, w1, nrm):
    return pl.pallas_call(
        _tc1_body,
        grid=(2, N // BLK),
        in_specs=[
            pl.BlockSpec((BLK, D), lambda p, i: (i, 0)),
            pl.BlockSpec((D, H), lambda p, i: (0, 0)),
            pl.BlockSpec((BLK, 2), lambda p, i: (i, 0)),
        ],
        out_specs=[
            pl.BlockSpec((BLK, H), lambda p, i: (p * i, 0)),
            pl.BlockSpec((8, H), lambda p, i: (0, 0)),
        ],
        out_shape=[
            jax.ShapeDtypeStruct((N, H), jnp.int16),
            jax.ShapeDtypeStruct((8, H), jnp.float32),
        ],
        scratch_shapes=[pltpu.VMEM((N, H), jnp.float32)],
    )(x, w1, nrm)


def _tc2_body(part_ref, pmax_ref, nrm_ref, b_ref, w_ref, q_ref, max_ref, ybuf_ref):
    p = pl.program_id(0)
    i = pl.program_id(1)

    @pl.when(p == 0)
    def _():
        agg = _dequant_agg(part_ref, pmax_ref)
        inrm = nrm_ref[:, 1:2]
        onrm = nrm_ref[:, 0:1]
        h = jnp.maximum(agg * inrm + b_ref[...], 0.0)
        y = onrm * jnp.dot(h, w_ref[...], preferred_element_type=jnp.float32)
        ybuf_ref[pl.ds(i * BLK, BLK), :] = y
        mb = _blockmax(y)

        @pl.when(i == 0)
        def _():
            max_ref[...] = mb

        @pl.when(i > 0)
        def _():
            max_ref[...] = jnp.maximum(max_ref[...], mb)

    @pl.when(p == 1)
    def _():
        _quantize(ybuf_ref, max_ref, i, q_ref)


def _tc2_call(part, pmax, nrm, b1, w2):
    return pl.pallas_call(
        _tc2_body,
        grid=(2, N // BLK),
        in_specs=[
            pl.BlockSpec((NC, BLK, H), lambda p, i: (0, i, 0)),
            pl.BlockSpec((8, H), lambda p, i: (0, 0)),
            pl.BlockSpec((BLK, 2), lambda p, i: (i, 0)),
            pl.BlockSpec((1, H), lambda p, i: (0, 0)),
            pl.BlockSpec((H, H), lambda p, i: (0, 0)),
        ],
        out_specs=[
            pl.BlockSpec((BLK, H), lambda p, i: (p * i, 0)),
            pl.BlockSpec((8, H), lambda p, i: (0, 0)),
        ],
        out_shape=[
            jax.ShapeDtypeStruct((N, H), jnp.int16),
            jax.ShapeDtypeStruct((8, H), jnp.float32),
        ],
        scratch_shapes=[pltpu.VMEM((N, H), jnp.float32)],
    )(part, pmax, nrm, b1, w2)


def _tc3_body(part_ref, pmax_ref, nrm_ref, b_ref, o_ref):
    agg = _dequant_agg(part_ref, pmax_ref)
    inrm = nrm_ref[:, 1:2]
    o_ref[...] = jnp.maximum(agg * inrm + b_ref[...], 0.0)


def _tc3_call(part, pmax, nrm, b2):
    return pl.pallas_call(
        _tc3_body,
        grid=(N // BLK,),
        in_specs=[
            pl.BlockSpec((NC, BLK, H), lambda i: (0, i, 0)),
            pl.BlockSpec((8, H), lambda i: (0, 0)),
            pl.BlockSpec((BLK, 2), lambda i: (i, 0)),
            pl.BlockSpec((1, H), lambda i: (0, 0)),
        ],
        out_specs=pl.BlockSpec((BLK, H), lambda i: (i, 0)),
        out_shape=jax.ShapeDtypeStruct((N, H), jnp.float32),
    )(part, pmax, nrm, b2)


def kernel(features, edge_index, W1, b1, W2, b2):
    aux = jnp.zeros((16,), jnp.int32).at[8].set(1)
    hist = _hist_call(edge_index, aux)
    nrm = _nrm_call(hist)
    q1, max1 = _tc1_call(features, W1, nrm)
    part1 = _scatter_call(q1, edge_index)
    q2, max2 = _tc2_call(part1, max1, nrm, b1.reshape(1, H), W2)
    part2 = _scatter_call(q2, edge_index)
    out = _tc3_call(part2, max2, nrm, b2.reshape(1, H))
    return out


# speedup vs baseline: 1.0356x; 1.0356x over previous
import functools

import jax
import jax.numpy as jnp
from jax import lax
from jax.experimental import pallas as pl
from jax.experimental.pallas import tpu as pltpu
from jax.experimental.pallas import tpu_sc as plsc

N = 10000
E = 320000
D = 128
H = 128

NC = 2
NS = 16
NW = NC * NS
EPT = E // NW
CH = 40
NCH = EPT // CH
ROWS_PT = N // NS
BLK = 2000
NBUF = 5
NFULL = (NCH // NBUF - 1) * NBUF


def _mesh():
    return plsc.VectorSubcoreMesh(
        core_axis_name="c", subcore_axis_name="s", num_cores=NC, num_subcores=NS
    )


_SC_PARAMS = pltpu.CompilerParams(
    needs_layout_passes=False, use_tc_tiling_on_sc=False
)


def _hist_body(ei_hbm, aux_hbm, hist_hbm, srcv, dstv, hist_o, hist_i, ridx, shd):
    cid = lax.axis_index("c")
    sid = lax.axis_index("s")
    wid = cid * NS + sid
    base = wid * EPT
    pltpu.sync_copy(ei_hbm.at[0, pl.ds(base, EPT)], srcv)
    pltpu.sync_copy(ei_hbm.at[1, pl.ds(base, EPT)], dstv)
    pltpu.sync_copy(aux_hbm, ridx)

    zeros16 = jnp.zeros((16,), jnp.float32)

    def zbody(i, c):
        for u in range(5):
            o = (i * 5 + u) * 16
            hist_o[0, pl.ds(o, 16)] = zeros16
            hist_i[0, pl.ds(o, 16)] = zeros16
        return c

    lax.fori_loop(0, N // 80, zbody, 0)
    @pl.when(sid < NS - 6)
    def _():
        row = sid // 5
        pltpu.sync_copy(
            hist_o.at[0, pl.ds(0, 1000)],
            shd.at[row, pl.ds((sid % 5) * 2000, 1000)],
        )
        pltpu.sync_copy(
            hist_i.at[0, pl.ds(0, 1000)],
            shd.at[row, pl.ds((sid % 5) * 2000 + 1000, 1000)],
        )

    plsc.subcore_barrier()

    ones16 = jnp.ones((16,), jnp.float32)
    zidx16 = jnp.zeros((16,), jnp.int32)

    def ebody(i, c):
        for u in range(5):
            o = (i * 5 + u) * 16
            s = srcv[pl.ds(o, 16)]
            plsc.addupdate_scatter(hist_o, [zidx16, s], ones16)
            d = dstv[pl.ds(o, 16)]
            plsc.addupdate_scatter(hist_i, [zidx16, d], ones16)
        return c

    lax.fori_loop(0, EPT // 80, ebody, 0)

    pltpu.sync_copy(hist_o, shd.at[ridx.at[pl.ds(0, 1)]], add=True)
    pltpu.sync_copy(hist_i, shd.at[ridx.at[pl.ds(8, 1)]], add=True)
    plsc.subcore_barrier()

    @pl.when(sid == 0)
    def _():
        pltpu.sync_copy(shd.at[0], hist_hbm.at[0, cid])
        pltpu.sync_copy(shd.at[1], hist_hbm.at[1, cid])


def _hist_call(edge_index, aux):
    f = pl.kernel(
        _hist_body,
        out_type=jax.ShapeDtypeStruct((2, NC, N), jnp.float32),
        mesh=_mesh(),
        scratch_types=[
            pltpu.VMEM((EPT,), jnp.int32),
            pltpu.VMEM((EPT,), jnp.int32),
            pltpu.VMEM((1, N), jnp.float32),
            pltpu.VMEM((1, N), jnp.float32),
            pltpu.VMEM((16,), jnp.int32),
            pltpu.VMEM_SHARED((2, N), jnp.float32),
        ],
        compiler_params=_SC_PARAMS,
    )
    return f(edge_index, aux)


def _scatter_body(y_hbm, ei_hbm, part_hbm, srcv, dstv, rows, *rest):
    sems = rest[:NBUF]
    dsem = rest[NBUF]
    zsem = rest[NBUF + 1]
    acc = rest[NBUF + 2]
    cid = lax.axis_index("c")
    sid = lax.axis_index("s")
    wid = cid * NS + sid
    base = wid * EPT
    r0 = sid * ROWS_PT
    pltpu.sync_copy(ei_hbm.at[0, pl.ds(base, EPT)], srcv)
    def stage(r, c):
        pltpu.async_copy(ei_hbm.at[1, pl.ds(base + r * CH, CH)], dstv.at[r], dsem)
        return c

    lax.fori_loop(0, NCH, stage, 0)
    zeros16 = jnp.zeros((16,), jnp.float32)

    def zb(i, c):
        for u in range(H // 16):
            rows[0, i, pl.ds(u * 16, 16)] = zeros16
        return c

    lax.fori_loop(0, CH, zb, 0)
    for t in range(ROWS_PT // CH):
        pltpu.async_copy(rows.at[0], acc.at[pl.ds(r0 + t * CH, CH)], zsem)
    pltpu.async_copy(
        rows.at[0, pl.ds(0, ROWS_PT % CH)],
        acc.at[pl.ds(r0 + (ROWS_PT // CH) * CH, ROWS_PT % CH)],
        zsem,
    )
    for t in range(ROWS_PT // CH):
        pltpu.make_async_copy(rows.at[0], acc.at[pl.ds(r0 + t * CH, CH)], zsem).wait()
    pltpu.make_async_copy(
        rows.at[0, pl.ds(0, ROWS_PT % CH)],
        acc.at[pl.ds(r0 + (ROWS_PT // CH) * CH, ROWS_PT % CH)],
        zsem,
    ).wait()

    def drain(r, c):
        pltpu.make_async_copy(
            ei_hbm.at[1, pl.ds(base + r * CH, CH)], dstv.at[r], dsem
        ).wait()
        return c

    lax.fori_loop(0, NCH, drain, 0)
    plsc.subcore_barrier()

    for b in range(NBUF):
        pltpu.async_copy(y_hbm.at[srcv.at[pl.ds(b * CH, CH)]], rows.at[b], sems[b])

    def body(g, c):
        for b in range(NBUF):
            j = g * NBUF + b
            pltpu.make_async_copy(
                y_hbm.at[srcv.at[pl.ds(j * CH, CH)]], rows.at[b], sems[b]
            ).wait()
            pltpu.sync_copy(rows.at[b], acc.at[dstv.at[j]], add=True)
            pltpu.async_copy(
                y_hbm.at[srcv.at[pl.ds((j + NBUF) * CH, CH)]], rows.at[b], sems[b]
            )
        return c

    lax.fori_loop(0, NCH // NBUF - 1, body, 0)
    for b in range(NBUF):
        j = NFULL + b
        pltpu.make_async_copy(
            y_hbm.at[srcv.at[pl.ds(j * CH, CH)]], rows.at[b], sems[b]
        ).wait()
        pltpu.sync_copy(rows.at[b], acc.at[dstv.at[j]], add=True)
    for j in range(NFULL + NBUF, NCH):
        b = j % NBUF
        pltpu.async_copy(y_hbm.at[srcv.at[pl.ds(j * CH, CH)]], rows.at[b], sems[b]).wait()
        pltpu.sync_copy(rows.at[b], acc.at[dstv.at[j]], add=True)

    plsc.subcore_barrier()
    pltpu.sync_copy(acc.at[pl.ds(r0, ROWS_PT)], part_hbm.at[cid, pl.ds(r0, ROWS_PT)])


def _scatter_call(y, edge_index):
    f = pl.kernel(
        _scatter_body,
        out_type=jax.ShapeDtypeStruct((NC, N, H), jnp.float32),
        mesh=_mesh(),
        scratch_types=[
            pltpu.VMEM((EPT,), jnp.int32),
            pltpu.VMEM((NCH, CH), jnp.int32),
            pltpu.VMEM((NBUF, CH, H), jnp.float32),
        ]
        + [pltpu.SemaphoreType.DMA] * (NBUF + 2)
        + [pltpu.VMEM_SHARED((N, H), jnp.float32)],
        compiler_params=_SC_PARAMS,
    )
    return f(y, edge_index)


def _nrm_body(hist_ref, nrm_ref):
    ones_w = jnp.ones((NC, 1), jnp.float32)
    dn = (((0,), (0,)), ((), ()))
    od = lax.dot_general(hist_ref[0], ones_w, dn, preferred_element_type=jnp.float32)
    idg = lax.dot_general(hist_ref[1], ones_w, dn, preferred_element_type=jnp.float32)
    onrm = lax.rsqrt(jnp.maximum(od, 1.0))
    inrm = lax.rsqrt(jnp.maximum(idg, 1.0))
    nrm_ref[...] = jnp.concatenate([onrm, inrm], axis=1)


def _nrm_call(hist):
    return pl.pallas_call(
        _nrm_body,
        grid=(1,),
        in_specs=[pl.BlockSpec((2, NC, N), lambda i: (0, 0, 0))],
        out_specs=pl.BlockSpec((N, 2), lambda i: (0, 0)),
        out_shape=jax.ShapeDtypeStruct((N, 2), jnp.float32),
    )(hist)


def _tc1_body(x_ref, w_ref, nrm_ref, y_ref):
    z = jnp.dot(x_ref[...], w_ref[...], preferred_element_type=jnp.float32)
    y_ref[...] = nrm_ref[:, 0:1] * z


def _tc1_call(x, w1, nrm):
    return pl.pallas_call(
        _tc1_body,
        grid=(N // BLK,),
        in_specs=[
            pl.BlockSpec((BLK, D), lambda i: (i, 0)),
            pl.BlockSpec((D, H), lambda i: (0, 0)),
            pl.BlockSpec((BLK, 2), lambda i: (i, 0)),
        ],
        out_specs=pl.BlockSpec((BLK, H), lambda i: (i, 0)),
        out_shape=jax.ShapeDtypeStruct((N, H), jnp.float32),
    )(x, w1, nrm)


def _tc2_body(part_ref, nrm_ref, b_ref, w_ref, y_ref):
    agg = part_ref[0] + part_ref[1]
    inrm = nrm_ref[:, 1:2]
    onrm = nrm_ref[:, 0:1]
    h = jnp.maximum(agg * inrm + b_ref[...], 0.0)
    y_ref[...] = onrm * jnp.dot(h, w_ref[...], preferred_element_type=jnp.float32)


def _tc2_call(part, nrm, b1, w2):
    return pl.pallas_call(
        _tc2_body,
        grid=(N // BLK,),
        in_specs=[
            pl.BlockSpec((NC, BLK, H), lambda i: (0, i, 0)),
            pl.BlockSpec((BLK, 2), lambda i: (i, 0)),
            pl.BlockSpec((1, H), lambda i: (0, 0)),
            pl.BlockSpec((H, H), lambda i: (0, 0)),
        ],
        out_specs=pl.BlockSpec((BLK, H), lambda i: (i, 0)),
        out_shape=jax.ShapeDtypeStruct((N, H), jnp.float32),
    )(part, nrm, b1, w2)


def _tc3_body(part_ref, nrm_ref, b_ref, o_ref):
    agg = part_ref[0] + part_ref[1]
    inrm = nrm_ref[:, 1:2]
    o_ref[...] = jnp.maximum(agg * inrm + b_ref[...], 0.0)


def _tc3_call(part, nrm, b2):
    return pl.pallas_call(
        _tc3_body,
        grid=(N // BLK,),
        in_specs=[
            pl.BlockSpec((NC, BLK, H), lambda i: (0, i, 0)),
            pl.BlockSpec((BLK, 2), lambda i: (i, 0)),
            pl.BlockSpec((1, H), lambda i: (0, 0)),
        ],
        out_specs=pl.BlockSpec((BLK, H), lambda i: (i, 0)),
        out_shape=jax.ShapeDtypeStruct((N, H), jnp.float32),
    )(part, nrm, b2)


def kernel(features, edge_index, W1, b1, W2, b2):
    aux = jnp.zeros((16,), jnp.int32).at[8].set(1)
    hist = _hist_call(edge_index, aux)
    nrm = _nrm_call(hist)
    y1 = _tc1_call(features, W1, nrm)
    part1 = _scatter_call(y1, edge_index)
    y2 = _tc2_call(part1, nrm, b1.reshape(1, H), W2)
    part2 = _scatter_call(y2, edge_index)
    out = _tc3_call(part2, nrm, b2.reshape(1, H))
    return out
